# baseline (speedup 1.0000x reference)
import functools

import jax
import jax.numpy as jnp
from jax import lax
from jax.experimental import pallas as pl
from jax.experimental.pallas import tpu as pltpu
from jax.experimental.pallas import tpu_sc as plsc

N_POS = 8192
D = 1024
NC = 2
NS = 16
NW = NC * NS


def _build(B: int, C: int, NBUF: int):
    b_per_w = B // NW
    nsteps = b_per_w // C
    assert nsteps % NBUF == 0 and nsteps >= 2 * NBUF
    mesh = plsc.VectorSubcoreMesh(core_axis_name="c", subcore_axis_name="s")

    @functools.partial(
        pl.kernel,
        out_type=jax.ShapeDtypeStruct((B, D), jnp.float32),
        mesh=mesh,
        scratch_types=[
            pltpu.VMEM((b_per_w,), jnp.int32),
            [pltpu.VMEM((C, D), jnp.float32)] * NBUF,
            [pltpu.SemaphoreType.DMA] * NBUF,
            [pltpu.SemaphoreType.DMA] * NBUF,
        ],
    )
    def gather_kernel(table_hbm, idx_hbm, out_hbm, idx_v, bufs, gsems, ssems):
        wid = lax.axis_index("s") * NC + lax.axis_index("c")
        base = wid * b_per_w
        pltpu.sync_copy(idx_hbm.at[pl.ds(base, b_per_w)], idx_v)

        def start_gather(i, b):
            off = pl.multiple_of(i * C, 8)
            pltpu.async_copy(table_hbm.at[idx_v.at[pl.ds(off, C)]], bufs[b], gsems[b])

        def wait_gather(b):
            pltpu.make_async_copy(
                table_hbm.at[idx_v.at[pl.ds(0, C)]], bufs[b], gsems[b]
            ).wait()

        def start_store(i, b):
            pltpu.async_copy(bufs[b], out_hbm.at[pl.ds(base + i * C, C)], ssems[b])

        def wait_store(b):
            pltpu.make_async_copy(
                bufs[b], out_hbm.at[pl.ds(base, C)], ssems[b]
            ).wait()

        for k in range(NBUF - 1):
            start_gather(k, k)

        @pl.loop(0, nsteps, step=NBUF)
        def _(g):
            for b in range(NBUF):
                k = g + b
                wait_gather(b)
                start_store(k, b)
                nb = (b + NBUF - 1) % NBUF

                @pl.when(k >= 1)
                def _():
                    wait_store(nb)

                @pl.when(k + NBUF - 1 < nsteps)
                def _():
                    start_gather(k + NBUF - 1, nb)

        wait_store((nsteps - 1) % NBUF)

    return gather_kernel


@jax.jit
def kernel(x, table):
    orig_shape = x.shape
    idx = x.reshape(-1).astype(jnp.int32)
    out = _build(idx.shape[0], 8, 8)(table, idx)
    return out.reshape(*orig_shape, D)
